# baseline (speedup 1.0000x reference)
import jax
import jax.numpy as jnp
from jax import lax
from jax.experimental import pallas as pl
from jax.experimental.pallas import tpu as pltpu
from jax.experimental.pallas import tpu_sc as plsc


_NC, _NS, _L = 2, 16, 16
_NW = _NC * _NS
_B = 4
_S = 2048
_D = 1024
_SC_B = 1
_SPW = _S // _NW
_CH = 16
_NSC = _SPW // _CH
_NITEM = _NSC * _SC_B
_RING = 3
_VPR = _D // _L


def _sc_body(x_hbm, p_hbm, o_hbm, bufp, bufs, semx, semo, semp):
    cid = lax.axis_index("c")
    sid = lax.axis_index("s")
    wid = sid * _NC + cid
    s0 = wid * _SPW

    cpp = pltpu.async_copy(p_hbm.at[pl.ds(s0, _SPW)], bufp, semp)

    def start_in(i):
        sc, b = divmod(i, _SC_B)
        return pltpu.async_copy(
            x_hbm.at[_B - _SC_B + b, pl.ds(s0 + sc * _CH, _CH)],
            bufs[i % _RING],
            semx[i % _RING],
        )

    cps = [None] * _NITEM
    cpo = [None] * _NITEM
    for i in range(min(_RING - 1, _NITEM)):
        cps[i] = start_in(i)
    cpp.wait()

    for i in range(_NITEM):
        sc, b = divmod(i, _SC_B)
        buf = bufs[i % _RING]
        cps[i].wait()

        @pl.loop(0, _CH)
        def _(r):
            pr = sc * _CH + r

            @plsc.parallel_loop(0, _VPR, unroll=8)
            def _(j):
                plsc.addupdate(
                    buf.at[r, pl.ds(j * _L, _L)],
                    bufp[pr, pl.ds(j * _L, _L)],
                )

        cpo[i] = pltpu.async_copy(
            buf, o_hbm.at[b, pl.ds(s0 + sc * _CH, _CH)], semo[i % _RING]
        )
        nxt = i + _RING - 1
        if nxt < _NITEM:
            if i >= 1:
                cpo[i - 1].wait()
            cps[nxt] = start_in(nxt)

    for i in range(max(0, _NITEM - _RING + 1), _NITEM):
        cpo[i].wait()


def _sc_call(inputs, pos_table):
    return pl.kernel(
        _sc_body,
        out_type=jax.ShapeDtypeStruct((_SC_B, _S, _D), inputs.dtype),
        mesh=plsc.VectorSubcoreMesh(core_axis_name="c", subcore_axis_name="s"),
        scratch_types=[
            pltpu.VMEM((_SPW, _D), jnp.float32),
            [pltpu.VMEM((_CH, _D), jnp.float32) for _ in range(_RING)],
            [pltpu.SemaphoreType.DMA for _ in range(_RING)],
            [pltpu.SemaphoreType.DMA for _ in range(_RING)],
            pltpu.SemaphoreType.DMA,
        ],
    )(inputs, pos_table)


def _tc_add_body(x_ref, p_ref, o_ref):
    o_ref[...] = x_ref[...] + p_ref[...]


def _tc_call(inputs, pos_table):
    batch, seq_len, d_model = inputs.shape
    tc_b = batch - _SC_B
    return pl.pallas_call(
        _tc_add_body,
        grid=(1, tc_b),
        in_specs=[
            pl.BlockSpec((1, seq_len, d_model), lambda i, j: (j, i, 0)),
            pl.BlockSpec((seq_len, d_model), lambda i, j: (i, 0)),
        ],
        out_specs=pl.BlockSpec((1, seq_len, d_model), lambda i, j: (j, i, 0)),
        out_shape=jax.ShapeDtypeStruct((tc_b, seq_len, d_model), inputs.dtype),
        compiler_params=pltpu.CompilerParams(
            dimension_semantics=("parallel", "parallel"),
        ),
    )(inputs, pos_table)


def kernel(inputs, pos_table):
    sc_out = _sc_call(inputs, pos_table)
    tc_out = _tc_call(inputs, pos_table)
    return jnp.concatenate([tc_out, sc_out], axis=0)


# speedup vs baseline: 2.5429x; 2.5429x over previous
import jax
import jax.numpy as jnp
from jax.experimental import pallas as pl
from jax.experimental.pallas import tpu as pltpu


_B = 4
_S = 2048
_D = 1024
_CH = 256
_NB = _S // _CH
_NITEM = _B * _NB
_PF = 4


def _body(x_hbm, p_hbm, o_hbm, bufs, pos_v, sem_in, sem_out, sem_p):
    cpp = pltpu.async_copy(p_hbm, pos_v, sem_p)

    def start_in(i):
        b, sc = divmod(i, _NB)
        return pltpu.async_copy(
            x_hbm.at[b, pl.ds(sc * _CH, _CH)], bufs.at[i % _NB], sem_in.at[i % _NB]
        )

    cps = [None] * _NITEM
    cpo = [None] * _NITEM
    for i in range(_PF):
        cps[i] = start_in(i)
    cpp.wait()

    for i in range(_NITEM):
        b, sc = divmod(i, _NB)
        k = i % _NB
        cps[i].wait()
        bufs[k] = bufs[k] + pos_v[pl.ds(sc * _CH, _CH), :]
        cpo[i] = pltpu.async_copy(
            bufs.at[k], o_hbm.at[b, pl.ds(sc * _CH, _CH)], sem_out.at[k]
        )
        j = i + _PF
        if j < _NITEM:
            if j >= _NB:
                cpo[j - _NB].wait()
            cps[j] = start_in(j)

    for i in range(_NITEM - _NB, _NITEM):
        cpo[i].wait()


def kernel(inputs, pos_table):
    return pl.pallas_call(
        _body,
        in_specs=[
            pl.BlockSpec(memory_space=pltpu.HBM),
            pl.BlockSpec(memory_space=pltpu.HBM),
        ],
        out_specs=pl.BlockSpec(memory_space=pltpu.HBM),
        out_shape=jax.ShapeDtypeStruct(inputs.shape, inputs.dtype),
        scratch_shapes=[
            pltpu.VMEM((_NB, _CH, _D), jnp.float32),
            pltpu.VMEM((_S, _D), jnp.float32),
            pltpu.SemaphoreType.DMA((_NB,)),
            pltpu.SemaphoreType.DMA((_NB,)),
            pltpu.SemaphoreType.DMA,
        ],
    )(inputs, pos_table)


# speedup vs baseline: 2.7709x; 1.0897x over previous
import jax
import jax.numpy as jnp
from jax.experimental import pallas as pl
from jax.experimental.pallas import tpu as pltpu


_SEQ_BLK = 2048


def _add_kernel(x_ref, p_ref, o_ref):
    o_ref[...] = x_ref[...] + p_ref[...]


def kernel(inputs, pos_table):
    batch, seq_len, d_model = inputs.shape
    n_seq = seq_len // _SEQ_BLK
    return pl.pallas_call(
        _add_kernel,
        grid=(n_seq, batch),
        in_specs=[
            pl.BlockSpec((1, _SEQ_BLK, d_model), lambda i, j: (j, i, 0)),
            pl.BlockSpec((_SEQ_BLK, d_model), lambda i, j: (i, 0)),
        ],
        out_specs=pl.BlockSpec((1, _SEQ_BLK, d_model), lambda i, j: (j, i, 0)),
        out_shape=jax.ShapeDtypeStruct(inputs.shape, inputs.dtype),
        compiler_params=pltpu.CompilerParams(
            dimension_semantics=("parallel", "parallel"),
        ),
    )(inputs, pos_table)
